# baseline (speedup 1.0000x reference)
import functools

import jax
import jax.numpy as jnp
from jax import lax
from jax.experimental import pallas as pl
from jax.experimental.pallas import tpu as pltpu
from jax.experimental.pallas import tpu_sc as plsc

B, N, D = 128, 8192, 64
NC, NS = 2, 16
NW = NC * NS
BPW = B // NW
ROWS_PW = BPW * N
ZR = 1024
NZ = ROWS_PW // ZR
BPS = 16
SCT = (B // NC) // BPS

_mesh = plsc.VectorSubcoreMesh(
    core_axis_name="c", subcore_axis_name="s", num_cores=NC, num_subcores=NS
)


@functools.partial(
    pl.kernel,
    out_type=jax.ShapeDtypeStruct((B * N, D), jnp.float32),
    mesh=_mesh,
    scratch_types=[
        pltpu.VMEM((B,), jnp.int32),
        pltpu.VMEM((BPS,), jnp.int32),
        pltpu.VMEM((BPS, D), jnp.float32),
        pltpu.VMEM((ZR, D), jnp.float32),
        pltpu.SemaphoreType.DMA,
        pltpu.SemaphoreType.DMA,
    ],
    compiler_params=pltpu.CompilerParams(use_tc_tiling_on_sc=False),
)
def _mask_rows(inp_hbm, tgt_hbm, zrc_hbm, out_hbm,
               tgt_v, idx_v, rows_v, zbuf, gsem, zsem):
    c = lax.axis_index("c")
    s = lax.axis_index("s")
    wid = c * NS + s

    pltpu.sync_copy(zrc_hbm, zbuf)

    zcopies = []
    for k in range(NZ):
        zcopies.append(
            pltpu.async_copy(
                zbuf, out_hbm.at[pl.ds(wid * ROWS_PW + k * ZR, ZR)], zsem
            )
        )

    @pl.when(s < SCT)
    def _gather():
        pltpu.sync_copy(tgt_hbm, tgt_v)
        bstart = c * (B // NC) + s * BPS
        lanes = lax.iota(jnp.int32, 16)
        t16 = tgt_v[pl.ds(bstart, BPS)]
        idx_v[...] = (bstart + lanes) * N + t16
        pltpu.async_copy(inp_hbm.at[idx_v], rows_v, gsem).wait()

    for cpy in zcopies:
        cpy.wait()
    plsc.subcore_barrier()

    @pl.when(s < SCT)
    def _scatter():
        pltpu.sync_copy(rows_v, out_hbm.at[idx_v])


def kernel(input, target):
    inp2 = input.reshape(B * N, D)
    tgt = target.astype(jnp.int32)
    zrc = jnp.zeros((ZR, D), jnp.float32)
    out = _mask_rows(inp2, tgt, zrc)
    return out.reshape(B, N, D)


# speedup vs baseline: 1.3318x; 1.3318x over previous
import functools

import jax
import jax.numpy as jnp
from jax import lax
from jax.experimental import pallas as pl
from jax.experimental.pallas import tpu as pltpu
from jax.experimental.pallas import tpu_sc as plsc

B, N, D = 128, 8192, 64
NC, NS = 2, 16
BPS = 16
SCT = (B // NC) // BPS

ZR = 8192
NZD = (B * N) // ZR

_mesh = plsc.VectorSubcoreMesh(
    core_axis_name="c", subcore_axis_name="s", num_cores=NC, num_subcores=NS
)


@functools.partial(
    pl.kernel,
    out_type=jax.ShapeDtypeStruct((B, D), jnp.float32),
    mesh=_mesh,
    scratch_types=[
        pltpu.VMEM((B,), jnp.int32),
        pltpu.VMEM((BPS,), jnp.int32),
        pltpu.VMEM((BPS, D), jnp.float32),
        pltpu.SemaphoreType.DMA,
    ],
    compiler_params=pltpu.CompilerParams(use_tc_tiling_on_sc=False),
)
def _gather_rows(inp_hbm, tgt_hbm, out_hbm, tgt_v, idx_v, rows_v, gsem):
    c = lax.axis_index("c")
    s = lax.axis_index("s")

    @pl.when(s < SCT)
    def _():
        pltpu.sync_copy(tgt_hbm, tgt_v)
        bstart = c * (B // NC) + s * BPS
        lanes = lax.iota(jnp.int32, 16)
        t16 = tgt_v[pl.ds(bstart, BPS)]
        idx_v[...] = (bstart + lanes) * N + t16
        pltpu.async_copy(inp_hbm.at[idx_v], rows_v, gsem).wait()
        pltpu.sync_copy(rows_v, out_hbm.at[pl.ds(bstart, BPS)])


def _place_body(tgt_ref, rows_ref, zrc_ref, out_hbm, zsem, psem):
    zcopies = []
    for i in range(NZD):
        zcopies.append(
            pltpu.async_copy(zrc_ref, out_hbm.at[pl.ds(i * ZR, ZR)], zsem)
        )
    for cpy in zcopies:
        cpy.wait()
    pcopies = []
    for b in range(B):
        t = tgt_ref[b]
        pcopies.append(
            pltpu.async_copy(
                rows_ref.at[pl.ds(b, 1)], out_hbm.at[pl.ds(b * N + t, 1)], psem
            )
        )
    for cpy in pcopies:
        cpy.wait()


_place = pl.pallas_call(
    _place_body,
    grid_spec=pltpu.PrefetchScalarGridSpec(
        num_scalar_prefetch=1,
        grid=(1,),
        in_specs=[
            pl.BlockSpec((B, D), lambda i, tgt: (0, 0)),
            pl.BlockSpec((ZR, D), lambda i, tgt: (0, 0)),
        ],
        out_specs=pl.BlockSpec(memory_space=pltpu.MemorySpace.HBM),
        scratch_shapes=[pltpu.SemaphoreType.DMA, pltpu.SemaphoreType.DMA],
    ),
    out_shape=jax.ShapeDtypeStruct((B * N, D), jnp.float32),
)


def kernel(input, target):
    inp2 = input.reshape(B * N, D)
    tgt = target.astype(jnp.int32)
    rows = _gather_rows(inp2, tgt)
    zrc = jnp.zeros((ZR, D), jnp.float32)
    out = _place(tgt, rows, zrc)
    return out.reshape(B, N, D)


# speedup vs baseline: 1.3336x; 1.0014x over previous
import functools

import jax
import jax.numpy as jnp
from jax import lax
from jax.experimental import pallas as pl
from jax.experimental.pallas import tpu as pltpu
from jax.experimental.pallas import tpu_sc as plsc

B, N, D = 128, 8192, 64
NC, NS = 2, 16
BPS = 16
SCT = (B // NC) // BPS

ZR = 8192
NZD = (B * N) // ZR

_mesh = plsc.VectorSubcoreMesh(
    core_axis_name="c", subcore_axis_name="s", num_cores=NC, num_subcores=NS
)


@functools.partial(
    pl.kernel,
    out_type=jax.ShapeDtypeStruct((B, 2 * D), jnp.float32),
    mesh=_mesh,
    scratch_types=[
        pltpu.VMEM((B,), jnp.int32),
        pltpu.VMEM((BPS,), jnp.int32),
        pltpu.VMEM((BPS, 2 * D), jnp.float32),
        pltpu.SemaphoreType.DMA,
    ],
)
def _gather_pairs(inp_hbm, tgt_hbm, out_hbm, tgt_v, idx_v, rows_v, gsem):
    c = lax.axis_index("c")
    s = lax.axis_index("s")

    @pl.when(s < SCT)
    def _():
        pltpu.sync_copy(tgt_hbm, tgt_v)
        bstart = c * (B // NC) + s * BPS
        lanes = lax.iota(jnp.int32, 16)
        t16 = tgt_v[pl.ds(bstart, BPS)]
        idx_v[...] = ((bstart + lanes) * N + t16) >> 1
        pltpu.async_copy(inp_hbm.at[idx_v], rows_v, gsem).wait()
        pltpu.sync_copy(rows_v, out_hbm.at[pl.ds(bstart, BPS)])


def _place_body(tgt_ref, pairs_ref, tvec_ref, zrc_ref, out_hbm, rows_v, zsem, psem):
    par = (tvec_ref[...] % 2) == 1
    rows_v[...] = jnp.where(par, pairs_ref[:, D:], pairs_ref[:, :D])
    zcopies = []
    for i in range(NZD):
        zcopies.append(
            pltpu.async_copy(zrc_ref, out_hbm.at[pl.ds(i * ZR, ZR)], zsem)
        )
    for cpy in zcopies:
        cpy.wait()
    pcopies = []
    for b in range(B):
        t = tgt_ref[b]
        pcopies.append(
            pltpu.async_copy(
                rows_v.at[pl.ds(b, 1)], out_hbm.at[pl.ds(b * N + t, 1)], psem
            )
        )
    for cpy in pcopies:
        cpy.wait()


_place = pl.pallas_call(
    _place_body,
    grid_spec=pltpu.PrefetchScalarGridSpec(
        num_scalar_prefetch=1,
        grid=(1,),
        in_specs=[
            pl.BlockSpec((B, 2 * D), lambda i, tgt: (0, 0)),
            pl.BlockSpec((B, 1), lambda i, tgt: (0, 0)),
            pl.BlockSpec((ZR, D), lambda i, tgt: (0, 0)),
        ],
        out_specs=pl.BlockSpec(memory_space=pltpu.MemorySpace.HBM),
        scratch_shapes=[
            pltpu.VMEM((B, D), jnp.float32),
            pltpu.SemaphoreType.DMA,
            pltpu.SemaphoreType.DMA,
        ],
    ),
    out_shape=jax.ShapeDtypeStruct((B * N, D), jnp.float32),
)


def kernel(input, target):
    inp2 = input.reshape(B * N // 2, 2 * D)
    tgt = target.astype(jnp.int32)
    pairs = _gather_pairs(inp2, tgt)
    zrc = jnp.zeros((ZR, D), jnp.float32)
    out = _place(tgt, pairs, tgt[:, None], zrc)
    return out.reshape(B, N, D)


# speedup vs baseline: 4.4561x; 3.3413x over previous
import functools

import jax
import jax.numpy as jnp
from jax import lax
from jax.experimental import pallas as pl
from jax.experimental.pallas import tpu as pltpu
from jax.experimental.pallas import tpu_sc as plsc

B, N, D = 128, 8192, 64
NC, NS = 2, 16
BPS = 16
SCT = (B // NC) // BPS
WAVE = 8

BN = 2048
NB = N // BN

_mesh = plsc.VectorSubcoreMesh(
    core_axis_name="c", subcore_axis_name="s", num_cores=NC, num_subcores=NS
)


@functools.partial(
    pl.kernel,
    out_type=jax.ShapeDtypeStruct((B, D, 128), jnp.float32),
    mesh=_mesh,
    scratch_types=[
        pltpu.VMEM((B,), jnp.int32),
        pltpu.VMEM((WAVE, D, 128), jnp.float32),
        pltpu.SemaphoreType.DMA,
        pltpu.SemaphoreType.DMA,
    ],
    compiler_params=pltpu.CompilerParams(needs_layout_passes=False),
)
def _gather_chunks(x_hbm, tgt_hbm, out_hbm, tgt_v, chunk_v, gsem, osem):
    c = lax.axis_index("c")
    s = lax.axis_index("s")

    @pl.when(s < SCT)
    def _():
        pltpu.sync_copy(tgt_hbm, tgt_v)
        w = c * SCT + s
        bstart = w * BPS
        lanes = lax.iota(jnp.int32, 16)
        t16 = tgt_v[pl.ds(bstart, BPS)]
        c16 = (t16 >> 7) << 7
        for wave in range(BPS // WAVE):
            copies = []
            for j in range(WAVE):
                jj = wave * WAVE + j
                c_j = jnp.sum(jnp.where(lanes == jj, c16, 0))
                c_j = pl.multiple_of(c_j, 128)
                copies.append(
                    pltpu.async_copy(
                        x_hbm.at[pl.ds((bstart + jj) * D, D),
                                 pl.ds(c_j, 128)],
                        chunk_v.at[j],
                        gsem,
                    )
                )
            for cpy in copies:
                cpy.wait()
            pltpu.sync_copy(
                chunk_v, out_hbm.at[pl.ds(bstart + wave * WAVE, WAVE)]
            )


def _place_body(tgt_ref, chunk_ref, out_ref):
    b = pl.program_id(0)
    k = pl.program_id(1)
    t = tgt_ref[b]
    tm = lax.rem(t, 128)
    rel = t - k * BN
    lane128 = lax.broadcasted_iota(jnp.int32, (D, 128), 1)
    col = jnp.sum(
        jnp.where(lane128 == tm, chunk_ref[0], 0.0), axis=1, keepdims=True
    )
    nidx = lax.broadcasted_iota(jnp.int32, (D, BN), 1)
    out_ref[0] = jnp.where(nidx == rel, col, 0.0)


_place = pl.pallas_call(
    _place_body,
    grid_spec=pltpu.PrefetchScalarGridSpec(
        num_scalar_prefetch=1,
        grid=(B, NB),
        in_specs=[
            pl.BlockSpec((1, D, 128), lambda b, k, tgt: (b, 0, 0)),
        ],
        out_specs=pl.BlockSpec((1, D, BN), lambda b, k, tgt: (b, 0, k)),
    ),
    out_shape=jax.ShapeDtypeStruct((B, D, N), jnp.float32),
)


def kernel(input, target):
    x2d = input.transpose(0, 2, 1).reshape(B * D, N)
    tgt = target.astype(jnp.int32)
    chunks = _gather_chunks(x2d, tgt)
    out3 = _place(tgt, chunks)
    return out3.transpose(0, 2, 1)


# speedup vs baseline: 10.7932x; 2.4221x over previous
import functools

import jax
import jax.numpy as jnp
from jax import lax
from jax.experimental import pallas as pl
from jax.experimental.pallas import tpu as pltpu
from jax.experimental.pallas import tpu_sc as plsc

B, N, D = 128, 8192, 64
NC, NS = 2, 16
BPS = 16
SCT = (B // NC) // BPS
WAVE = 8

_mesh = plsc.VectorSubcoreMesh(
    core_axis_name="c", subcore_axis_name="s", num_cores=NC, num_subcores=NS
)


@functools.partial(
    pl.kernel,
    out_type=jax.ShapeDtypeStruct((B, D, 128), jnp.float32),
    mesh=_mesh,
    scratch_types=[
        pltpu.VMEM((B,), jnp.int32),
        pltpu.VMEM((WAVE, D, 128), jnp.float32),
        pltpu.SemaphoreType.DMA,
        pltpu.SemaphoreType.DMA,
    ],
    compiler_params=pltpu.CompilerParams(needs_layout_passes=False),
)
def _gather_chunks(x_hbm, tgt_hbm, out_hbm, tgt_v, chunk_v, gsem, osem):
    c = lax.axis_index("c")
    s = lax.axis_index("s")

    @pl.when(s < SCT)
    def _():
        pltpu.sync_copy(tgt_hbm, tgt_v)
        w = c * SCT + s
        bstart = w * BPS
        lanes = lax.iota(jnp.int32, 16)
        t16 = tgt_v[pl.ds(bstart, BPS)]
        c16 = (t16 >> 7) << 7
        for wave in range(BPS // WAVE):
            copies = []
            for j in range(WAVE):
                jj = wave * WAVE + j
                c_j = jnp.sum(jnp.where(lanes == jj, c16, 0))
                c_j = pl.multiple_of(c_j, 128)
                copies.append(
                    pltpu.async_copy(
                        x_hbm.at[pl.ds((bstart + jj) * D, D),
                                 pl.ds(c_j, 128)],
                        chunk_v.at[j],
                        gsem,
                    )
                )
            for cpy in copies:
                cpy.wait()
            pltpu.sync_copy(
                chunk_v, out_hbm.at[pl.ds(bstart + wave * WAVE, WAVE)]
            )


def _place_body(tgt_ref, chunk_ref, tvec_ref, zrc_ref, out_hbm,
                strips_v, zsem, psem):
    tm = jnp.reshape(lax.rem(tvec_ref[...], 128), (B, 1, 1))
    lane = lax.broadcasted_iota(jnp.int32, (B, D, 128), 2)
    strips_v[...] = jnp.where(lane == tm, chunk_ref[...], 0.0)
    zcopies = []
    for b in range(B):
        zcopies.append(pltpu.async_copy(zrc_ref, out_hbm.at[b], zsem))
    for cpy in zcopies:
        cpy.wait()
    pcopies = []
    for b in range(B):
        c0 = pl.multiple_of((tgt_ref[b] >> 7) << 7, 128)
        pcopies.append(
            pltpu.async_copy(
                strips_v.at[b], out_hbm.at[b, :, pl.ds(c0, 128)], psem
            )
        )
    for cpy in pcopies:
        cpy.wait()


_place = pl.pallas_call(
    _place_body,
    grid_spec=pltpu.PrefetchScalarGridSpec(
        num_scalar_prefetch=1,
        grid=(1,),
        in_specs=[
            pl.BlockSpec((B, D, 128), lambda i, tgt: (0, 0, 0)),
            pl.BlockSpec((B, 1), lambda i, tgt: (0, 0)),
            pl.BlockSpec((D, N), lambda i, tgt: (0, 0)),
        ],
        out_specs=pl.BlockSpec(memory_space=pltpu.MemorySpace.HBM),
        scratch_shapes=[
            pltpu.VMEM((B, D, 128), jnp.float32),
            pltpu.SemaphoreType.DMA,
            pltpu.SemaphoreType.DMA,
        ],
    ),
    out_shape=jax.ShapeDtypeStruct((B, D, N), jnp.float32),
)


def kernel(input, target):
    x2d = input.transpose(0, 2, 1).reshape(B * D, N)
    tgt = target.astype(jnp.int32)
    chunks = _gather_chunks(x2d, tgt)
    zrc = jnp.zeros((D, N), jnp.float32)
    out3 = _place(tgt, chunks, tgt[:, None], zrc)
    return out3.transpose(0, 2, 1)


# speedup vs baseline: 11.3543x; 1.0520x over previous
import functools

import jax
import jax.numpy as jnp
from jax import lax
from jax.experimental import pallas as pl
from jax.experimental.pallas import tpu as pltpu
from jax.experimental.pallas import tpu_sc as plsc

B, N, D = 128, 8192, 64
NC, NS = 2, 16
NW = NC * NS
BPW = B // NW

ZB = 2

_mesh = plsc.VectorSubcoreMesh(
    core_axis_name="c", subcore_axis_name="s", num_cores=NC, num_subcores=NS
)


@functools.partial(
    pl.kernel,
    out_type=jax.ShapeDtypeStruct((B, D, 128), jnp.float32),
    mesh=_mesh,
    scratch_types=[
        pltpu.VMEM((B,), jnp.int32),
        pltpu.VMEM((BPW, D, 128), jnp.float32),
        pltpu.SemaphoreType.DMA,
    ],
    compiler_params=pltpu.CompilerParams(needs_layout_passes=False),
)
def _gather_chunks(x_hbm, tgt_hbm, out_hbm, tgt_v, chunk_v, gsem):
    c = lax.axis_index("c")
    s = lax.axis_index("s")
    w = c * NS + s
    bstart = w * BPW

    pltpu.sync_copy(tgt_hbm, tgt_v)
    lanes = lax.iota(jnp.int32, 16)
    win = (w // 4) * 16
    off = (w % 4) * BPW
    t16 = tgt_v[pl.ds(win, 16)]
    c16 = (t16 >> 7) << 7
    copies = []
    for j in range(BPW):
        c_j = jnp.sum(jnp.where(lanes == off + j, c16, 0))
        c_j = pl.multiple_of(c_j, 128)
        copies.append(
            pltpu.async_copy(
                x_hbm.at[pl.ds((bstart + j) * D, D), pl.ds(c_j, 128)],
                chunk_v.at[j],
                gsem,
            )
        )
    for cpy in copies:
        cpy.wait()
    pltpu.sync_copy(chunk_v, out_hbm.at[pl.ds(bstart, BPW)])


def _place_body(tgt_ref, chunk_ref, tvec_ref, zrc_ref, out_hbm,
                strips_v, zsem0, zsem1, psem):
    tm = jnp.reshape(lax.rem(tvec_ref[...], 128), (B, 1, 1))
    lane = lax.broadcasted_iota(jnp.int32, (B, D, 128), 2)
    strips_v[...] = jnp.where(lane == tm, chunk_ref[...], 0.0)
    zsems = [zsem0, zsem1]
    zcopies = []
    for i in range(B // ZB):
        zcopies.append(
            pltpu.async_copy(
                zrc_ref, out_hbm.at[pl.ds(i * ZB, ZB)], zsems[i % 2]
            )
        )
    for cpy in zcopies:
        cpy.wait()
    pcopies = []
    for b in range(B):
        c0 = pl.multiple_of((tgt_ref[b] >> 7) << 7, 128)
        pcopies.append(
            pltpu.async_copy(
                strips_v.at[b], out_hbm.at[b, :, pl.ds(c0, 128)], psem
            )
        )
    for cpy in pcopies:
        cpy.wait()


_place = pl.pallas_call(
    _place_body,
    grid_spec=pltpu.PrefetchScalarGridSpec(
        num_scalar_prefetch=1,
        grid=(1,),
        in_specs=[
            pl.BlockSpec((B, D, 128), lambda i, tgt: (0, 0, 0)),
            pl.BlockSpec((B, 1), lambda i, tgt: (0, 0)),
            pl.BlockSpec((ZB, D, N), lambda i, tgt: (0, 0, 0)),
        ],
        out_specs=pl.BlockSpec(memory_space=pltpu.MemorySpace.HBM),
        scratch_shapes=[
            pltpu.VMEM((B, D, 128), jnp.float32),
            pltpu.SemaphoreType.DMA,
            pltpu.SemaphoreType.DMA,
            pltpu.SemaphoreType.DMA,
        ],
    ),
    out_shape=jax.ShapeDtypeStruct((B, D, N), jnp.float32),
)


def kernel(input, target):
    x2d = input.transpose(0, 2, 1).reshape(B * D, N)
    tgt = target.astype(jnp.int32)
    chunks = _gather_chunks(x2d, tgt)
    zrc = jnp.zeros((ZB, D, N), jnp.float32)
    out3 = _place(tgt, chunks, tgt[:, None], zrc)
    return out3.transpose(0, 2, 1)
